# baseline (speedup 1.0000x reference)
import jax
import jax.numpy as jnp
from jax import lax
from jax.experimental import pallas as pl
from jax.experimental.pallas import tpu as pltpu

T, D, H, O, E, K = 2048, 1024, 2048, 1024, 8, 2
BLK = 256
NB = (T * K) // BLK + E
P = NB * BLK
EP = 128

def _router_body(x_ref, wr_ref, br_ref, tw_ref, ti_ref, sm_ref):
    logits = jnp.dot(x_ref[...], wr_ref[...],
                     preferred_element_type=jnp.float32) + br_ref[...]
    idx = lax.broadcasted_iota(jnp.int32, logits.shape, 1)
    big = jnp.int32(1 << 30)
    m1 = jnp.max(logits, axis=-1, keepdims=True)
    i1 = jnp.min(jnp.where(logits == m1, idx, big), axis=-1, keepdims=True)
    l2 = jnp.where(idx == i1, -jnp.inf, logits)
    m2 = jnp.max(l2, axis=-1, keepdims=True)
    i2 = jnp.min(jnp.where(l2 == m2, idx, big), axis=-1, keepdims=True)
    den = jnp.sum(jnp.exp(logits - m1), axis=-1, keepdims=True)
    w1 = jnp.exp(m1 - m1) / den
    w2 = jnp.exp(m2 - m1) / den
    tw_ref[...] = jnp.where(idx == 0, w1, jnp.where(idx == 1, w2, 0.0))
    ti_ref[...] = jnp.where(idx == 0, i1, jnp.where(idx == 1, i2, 0))

    oh = (idx == i1).astype(jnp.int32) + (idx == i2).astype(jnp.int32)
    counts = jnp.sum(oh, axis=0, keepdims=True)
    nb = (counts + BLK - 1) // BLK
    jl = idx[0:1, :]
    em = jnp.zeros_like(jl)
    bvec = jnp.zeros_like(jl)
    bend_e = jnp.int32(0)
    for e in range(E):
        bend_e = bend_e + nb[0, e]
        em = em + (jl >= bend_e).astype(jnp.int32)
        bvec = bvec + (jl == 64 + e).astype(jnp.int32) * bend_e
    u = bend_e
    in_em = (jl < 32).astype(jnp.int32)
    in_va = ((jl >= 32) & (jl < 64)).astype(jnp.int32)
    sm_ref[...] = (in_em * jnp.minimum(em, E - 1)
                   + in_va * (jl - 32 < u).astype(jnp.int32)
                   + bvec)


def _router(x, Wr, br):
    wr_p = jnp.zeros((D, EP), jnp.float32).at[:, :E].set(Wr)
    br_p = jnp.full((1, EP), -1e30, jnp.float32).at[0, :E].set(br)
    tw, ti, sm = pl.pallas_call(
        _router_body,
        out_shape=(jax.ShapeDtypeStruct((T, EP), jnp.float32),
                   jax.ShapeDtypeStruct((T, EP), jnp.int32),
                   jax.ShapeDtypeStruct((1, EP), jnp.int32)),
    )(x, wr_p, br_p)
    return tw[:, :K].reshape(-1), ti[:, :K].reshape(-1), sm[0]


def _gemm_body(s_ref, xs_ref, w1_ref, b1_ref, w2_ref, b2_ref, out_ref):
    i = pl.program_id(0)

    @pl.when(s_ref[32 + i] == 1)
    def _():
        h = jnp.dot(xs_ref[...], w1_ref[0],
                    preferred_element_type=jnp.float32) + b1_ref[0]
        h = jnp.maximum(h, 0.0)
        y = jnp.dot(h, w2_ref[0],
                    preferred_element_type=jnp.float32) + b2_ref[0]
        out_ref[...] = y


def _grouped_gemm(smap, xs, W1, b1, W2, b2):
    grid_spec = pltpu.PrefetchScalarGridSpec(
        num_scalar_prefetch=1,
        grid=(NB,),
        in_specs=[
            pl.BlockSpec((BLK, D), lambda i, s: (i, 0)),
            pl.BlockSpec((1, D, H), lambda i, s: (s[i], 0, 0)),
            pl.BlockSpec((1, 1, H), lambda i, s: (s[i], 0, 0)),
            pl.BlockSpec((1, H, O), lambda i, s: (s[i], 0, 0)),
            pl.BlockSpec((1, 1, O), lambda i, s: (s[i], 0, 0)),
        ],
        out_specs=pl.BlockSpec((BLK, O), lambda i, s: (i, 0)),
    )
    return pl.pallas_call(
        _gemm_body,
        grid_spec=grid_spec,
        out_shape=jax.ShapeDtypeStruct((P, O), jnp.float32),
        compiler_params=pltpu.CompilerParams(
            dimension_semantics=("arbitrary",)),
    )(smap, xs, W1, b1[:, None, :], W2, b2[:, None, :])


def kernel(x, Wr, br, W1, b1, W2, b2):
    w_flat, eid_flat, sm = _router(x, Wr, br)
    onehot = (eid_flat[:, None] ==
              jnp.arange(E, dtype=jnp.int32)[None, :]).astype(jnp.int32)
    rank = jnp.take_along_axis(jnp.cumsum(onehot, axis=0) - onehot,
                               eid_flat[:, None], axis=1)[:, 0]
    bstart = jnp.concatenate([jnp.zeros((1,), jnp.int32), sm[64:64 + E]])
    dest = bstart[eid_flat] * BLK + rank
    pairtok = jnp.arange(T * K, dtype=jnp.int32) // K
    tok_sorted = (jnp.arange(P, dtype=jnp.int32) % T).at[dest].set(pairtok)
    xs = x[tok_sorted]
    ys = _grouped_gemm(sm[:64], xs, W1, b1, W2, b2)
    sel = ys[dest].reshape(T, K, O)
    return jnp.sum(sel * w_flat.reshape(T, K, 1), axis=1)


# speedup vs baseline: 1.1374x; 1.1374x over previous
import jax
import jax.numpy as jnp
from jax import lax
from jax.experimental import pallas as pl
from jax.experimental.pallas import tpu as pltpu

T, D, H, O, E, K = 2048, 1024, 2048, 1024, 8, 2
BLK = 256
NB = (T * K) // BLK + E
P = NB * BLK
EP = 128

def _router_body(x_ref, wr_ref, br_ref, tw_ref, ti_ref, sm_ref):
    logits = jnp.dot(x_ref[...], wr_ref[...],
                     preferred_element_type=jnp.float32) + br_ref[...]
    idx = lax.broadcasted_iota(jnp.int32, logits.shape, 1)
    big = jnp.int32(1 << 30)
    m1 = jnp.max(logits, axis=-1, keepdims=True)
    i1 = jnp.min(jnp.where(logits == m1, idx, big), axis=-1, keepdims=True)
    l2 = jnp.where(idx == i1, -jnp.inf, logits)
    m2 = jnp.max(l2, axis=-1, keepdims=True)
    i2 = jnp.min(jnp.where(l2 == m2, idx, big), axis=-1, keepdims=True)
    den = jnp.sum(jnp.exp(logits - m1), axis=-1, keepdims=True)
    w1 = jnp.exp(m1 - m1) / den
    w2 = jnp.exp(m2 - m1) / den
    tw_ref[...] = jnp.where(idx == 0, w1, jnp.where(idx == 1, w2, 0.0))
    ti_ref[...] = jnp.where(idx == 0, i1, jnp.where(idx == 1, i2, 0))

    oh = (idx == i1).astype(jnp.int32) + (idx == i2).astype(jnp.int32)
    counts = jnp.sum(oh, axis=0, keepdims=True)
    nb = (counts + BLK - 1) // BLK
    jl = idx[0:1, :]
    em = jnp.zeros_like(jl)
    bvec = jnp.zeros_like(jl)
    bend_e = jnp.int32(0)
    for e in range(E):
        bend_e = bend_e + nb[0, e]
        em = em + (jl >= bend_e).astype(jnp.int32)
        bvec = bvec + (jl == 64 + e).astype(jnp.int32) * bend_e
    u = bend_e
    in_em = (jl < 32).astype(jnp.int32)
    in_va = ((jl >= 32) & (jl < 64)).astype(jnp.int32)
    sm_ref[...] = (in_em * jnp.minimum(em, E - 1)
                   + in_va * (jl - 32 < u).astype(jnp.int32)
                   + bvec)


def _router(x, Wr, br):
    wr_p = jnp.zeros((D, EP), jnp.float32).at[:, :E].set(Wr)
    br_p = jnp.full((1, EP), -1e30, jnp.float32).at[0, :E].set(br)
    tw, ti, sm = pl.pallas_call(
        _router_body,
        out_shape=(jax.ShapeDtypeStruct((T, EP), jnp.float32),
                   jax.ShapeDtypeStruct((T, EP), jnp.int32),
                   jax.ShapeDtypeStruct((1, EP), jnp.int32)),
    )(x, wr_p, br_p)
    return tw[:, :K].reshape(-1), ti[:, :K].reshape(-1), sm[0]


def _gemm_body(s_ref, xs_ref, w1_ref, b1_ref, w2_ref, b2_ref, out_ref):
    i = pl.program_id(0)

    @pl.when(s_ref[32 + i] == 1)
    def _():
        h = jnp.dot(xs_ref[...], w1_ref[0],
                    preferred_element_type=jnp.float32) + b1_ref[0]
        h = jnp.maximum(h, 0.0)
        y = jnp.dot(h, w2_ref[0],
                    preferred_element_type=jnp.float32) + b2_ref[0]
        out_ref[...] = y


def _grouped_gemm(smap, xs, W1, b1, W2, b2):
    grid_spec = pltpu.PrefetchScalarGridSpec(
        num_scalar_prefetch=1,
        grid=(NB,),
        in_specs=[
            pl.BlockSpec((BLK, D), lambda i, s: (i, 0)),
            pl.BlockSpec((1, D, H), lambda i, s: (s[i], 0, 0)),
            pl.BlockSpec((1, 1, H), lambda i, s: (s[i], 0, 0)),
            pl.BlockSpec((1, H, O), lambda i, s: (s[i], 0, 0)),
            pl.BlockSpec((1, 1, O), lambda i, s: (s[i], 0, 0)),
        ],
        out_specs=pl.BlockSpec((BLK, O), lambda i, s: (i, 0)),
    )
    return pl.pallas_call(
        _gemm_body,
        grid_spec=grid_spec,
        out_shape=jax.ShapeDtypeStruct((P, O), jnp.float32),
        compiler_params=pltpu.CompilerParams(
            dimension_semantics=("arbitrary",)),
    )(smap, xs, W1, b1[:, None, :], W2, b2[:, None, :])


def kernel(x, Wr, br, W1, b1, W2, b2):
    w_flat, eid_flat, sm = _router(x, Wr, br)
    onehot = (eid_flat[:, None] ==
              jnp.arange(E, dtype=jnp.int32)[None, :]).astype(jnp.int32)
    rank = jnp.take_along_axis(jnp.cumsum(onehot, axis=0) - onehot,
                               eid_flat[:, None], axis=1)[:, 0]
    bstart = jnp.concatenate([jnp.zeros((1,), jnp.int32), sm[64:64 + E]])
    dest = bstart[eid_flat] * BLK + rank
    pairtok = jnp.arange(T * K, dtype=jnp.int32) // K
    tok_sorted = (jnp.arange(P, dtype=jnp.int32) % T).at[dest].set(pairtok)
    xs = x[tok_sorted]
    ys = _grouped_gemm(sm[:64], xs, W1, b1, W2, b2)
    pp = dest.reshape(T, K)
    ww = w_flat.reshape(T, K)
    return ys[pp[:, 0]] * ww[:, :1] + ys[pp[:, 1]] * ww[:, 1:]
